# baseline (speedup 1.0000x reference)
import functools

import jax
import jax.numpy as jnp
from jax import lax
from jax.experimental import pallas as pl
from jax.experimental.pallas import tpu as pltpu
from jax.experimental.pallas import tpu_sc as plsc

EMB = 128
LANES = 16
COLS = EMB // LANES


def _tables_body(emb_ref, w_ref, b1_ref, b2_ref):
    e = emb_ref[...]
    b1_ref[...] = jnp.dot(e, w_ref[0:EMB, :], preferred_element_type=jnp.float32)
    b2_ref[...] = jnp.dot(e, w_ref[EMB:2 * EMB, :], preferred_element_type=jnp.float32)


def _y_body(rbf_ref, wr_ref, br_ref, w3_ref, b_ref, y_ref):
    t = jnp.dot(rbf_ref[...], wr_ref[...], preferred_element_type=jnp.float32)
    t = t + br_ref[...]
    s = t / (1.0 + jnp.exp(-t))
    y_ref[...] = jnp.dot(s, w3_ref[...], preferred_element_type=jnp.float32) + b_ref[...]


def _sc_body(n_chunks, chunk, b1_hbm, b2_hbm, z_hbm, ii_hbm, ij_hbm, y_hbm,
             out_hbm, b1_v, b2_v, z_v, ii_v, ij_v, ybuf):
    n_atoms = z_hbm.shape[0]
    wid = lax.axis_index("s") * 2 + lax.axis_index("c")
    base = wid * (n_chunks * chunk)

    pltpu.sync_copy(b1_hbm, b1_v)
    pltpu.sync_copy(b2_hbm, b2_v)
    pltpu.sync_copy(z_hbm, z_v)

    def chunk_body(c, _):
        off = base + c * chunk
        pltpu.sync_copy(ii_hbm.at[pl.ds(off, chunk)], ii_v)
        pltpu.sync_copy(ij_hbm.at[pl.ds(off, chunk)], ij_v)
        pltpu.sync_copy(y_hbm.at[pl.ds(off, chunk)], ybuf)

        def group_body(g, _):
            iv_i = ii_v[pl.ds(g * LANES, LANES)]
            iv_j = ij_v[pl.ds(g * LANES, LANES)]
            zi_vec = plsc.load_gather(z_v, [iv_i])
            zj_vec = plsc.load_gather(z_v, [iv_j])
            for l in range(LANES):
                zi = zi_vec[l]
                zj = zj_vec[l]
                e = g * LANES + l
                for k in range(COLS):
                    sl = pl.ds(k * LANES, LANES)
                    acc = ybuf[e, sl] + b1_v[zi, sl] + b2_v[zj, sl]
                    ybuf[e, sl] = acc / (1.0 + jnp.exp(-acc))
            return 0

        lax.fori_loop(0, chunk // LANES, group_body, 0)
        pltpu.sync_copy(ybuf, out_hbm.at[pl.ds(off, chunk)])
        return 0

    lax.fori_loop(0, n_chunks, chunk_body, 0)


def kernel(Z, rbf, idnb_i, idnb_j, embeddings, W_rbf, b_rbf, W, b):
    n_edges = rbf.shape[0]
    n_atoms = Z.shape[0]
    nv = embeddings.shape[0]

    nv_pad = ((nv + 7) // 8) * 8
    emb_p = jnp.pad(embeddings, ((0, nv_pad - nv), (0, 0)))
    b1, b2 = pl.pallas_call(
        _tables_body,
        out_shape=(
            jax.ShapeDtypeStruct((nv_pad, EMB), jnp.float32),
            jax.ShapeDtypeStruct((nv_pad, EMB), jnp.float32),
        ),
    )(emb_p, W)

    blk = 2000
    grid = n_edges // blk
    nr = rbf.shape[1]
    y = pl.pallas_call(
        _y_body,
        grid=(grid,),
        in_specs=[
            pl.BlockSpec((blk, nr), lambda i: (i, 0)),
            pl.BlockSpec((nr, EMB), lambda i: (0, 0)),
            pl.BlockSpec((1, EMB), lambda i: (0, 0)),
            pl.BlockSpec((EMB, EMB), lambda i: (0, 0)),
            pl.BlockSpec((1, EMB), lambda i: (0, 0)),
        ],
        out_specs=pl.BlockSpec((blk, EMB), lambda i: (i, 0)),
        out_shape=jax.ShapeDtypeStruct((n_edges, EMB), jnp.float32),
    )(rbf, W_rbf, b_rbf.reshape(1, EMB), W[2 * EMB:3 * EMB, :],
      b.reshape(1, EMB))

    n_workers = 32
    per_worker = n_edges // n_workers
    chunk = 400
    while per_worker % chunk or chunk % 8:
        chunk -= 8
    n_chunks = per_worker // chunk

    mesh = plsc.VectorSubcoreMesh(core_axis_name="c", subcore_axis_name="s",
                                  num_cores=2, num_subcores=16)
    sc = pl.kernel(
        functools.partial(_sc_body, n_chunks, chunk),
        out_type=jax.ShapeDtypeStruct((n_edges, EMB), jnp.float32),
        mesh=mesh,
        scratch_types=[
            pltpu.VMEM((nv_pad, EMB), jnp.float32),
            pltpu.VMEM((nv_pad, EMB), jnp.float32),
            pltpu.VMEM((n_atoms,), jnp.int32),
            pltpu.VMEM((chunk,), jnp.int32),
            pltpu.VMEM((chunk,), jnp.int32),
            pltpu.VMEM((chunk, EMB), jnp.float32),
        ],
        compiler_params=pltpu.CompilerParams(needs_layout_passes=False),
    )
    return sc(b1, b2, Z.astype(jnp.int32), idnb_i.astype(jnp.int32),
              idnb_j.astype(jnp.int32), y)


# speedup vs baseline: 2.7018x; 2.7018x over previous
import functools

import jax
import jax.numpy as jnp
from jax import lax
from jax.experimental import pallas as pl
from jax.experimental.pallas import tpu as pltpu
from jax.experimental.pallas import tpu_sc as plsc

EMB = 128
LANES = 16
COLS = EMB // LANES


def _tables_body(emb_ref, w_ref, b1_ref, b2_ref):
    e = emb_ref[...]
    b1_ref[...] = jnp.dot(e, w_ref[0:EMB, :], preferred_element_type=jnp.float32)
    b2_ref[...] = jnp.dot(e, w_ref[EMB:2 * EMB, :], preferred_element_type=jnp.float32)


def _y_body(rbf_ref, wr_ref, br_ref, w3_ref, b_ref, y_ref):
    t = jnp.dot(rbf_ref[...], wr_ref[...], preferred_element_type=jnp.float32)
    t = t + br_ref[...]
    s = t / (1.0 + jnp.exp(-t))
    y_ref[...] = jnp.dot(s, w3_ref[...], preferred_element_type=jnp.float32) + b_ref[...]


def _sc_body(n_chunks, chunk, b1_hbm, b2_hbm, z_hbm, ii_hbm, ij_hbm, y_hbm,
             out_hbm, z_v, ii_v, ij_v, zi_v, zj_v, g1buf, g2buf, ybuf, obuf,
             sem1, sem2):
    wid = lax.axis_index("s") * 2 + lax.axis_index("c")
    base = wid * (n_chunks * chunk)

    pltpu.sync_copy(z_hbm, z_v)

    def chunk_body(c, _):
        off = base + c * chunk
        pltpu.sync_copy(ii_hbm.at[pl.ds(off, chunk)], ii_v)
        pltpu.sync_copy(ij_hbm.at[pl.ds(off, chunk)], ij_v)

        @plsc.parallel_loop(0, chunk // LANES)
        def index_body(g):
            sl = pl.ds(g * LANES, LANES)
            zi_v[sl] = plsc.load_gather(z_v, [ii_v[sl]])
            zj_v[sl] = plsc.load_gather(z_v, [ij_v[sl]])

        cp1 = pltpu.make_async_copy(b1_hbm.at[zi_v], g1buf, sem1)
        cp2 = pltpu.make_async_copy(b2_hbm.at[zj_v], g2buf, sem2)
        cp1.start()
        cp2.start()
        pltpu.sync_copy(y_hbm.at[pl.ds(off, chunk)], ybuf)
        cp1.wait()
        cp2.wait()

        @plsc.parallel_loop(0, chunk, unroll=2)
        def row_body(r):
            for k in range(COLS):
                sl = pl.ds(k * LANES, LANES)
                acc = ybuf[r, sl] + g1buf[r, sl] + g2buf[r, sl]
                obuf[r, sl] = acc / (1.0 + jnp.exp(-acc))

        pltpu.sync_copy(obuf, out_hbm.at[pl.ds(off, chunk)])
        return 0

    lax.fori_loop(0, n_chunks, chunk_body, 0)


def kernel(Z, rbf, idnb_i, idnb_j, embeddings, W_rbf, b_rbf, W, b):
    n_edges = rbf.shape[0]
    n_atoms = Z.shape[0]
    nv = embeddings.shape[0]

    nv_pad = ((nv + 7) // 8) * 8
    emb_p = jnp.pad(embeddings, ((0, nv_pad - nv), (0, 0)))
    b1, b2 = pl.pallas_call(
        _tables_body,
        out_shape=(
            jax.ShapeDtypeStruct((nv_pad, EMB), jnp.float32),
            jax.ShapeDtypeStruct((nv_pad, EMB), jnp.float32),
        ),
    )(emb_p, W)

    blk = 2000
    grid = n_edges // blk
    nr = rbf.shape[1]
    y = pl.pallas_call(
        _y_body,
        grid=(grid,),
        in_specs=[
            pl.BlockSpec((blk, nr), lambda i: (i, 0)),
            pl.BlockSpec((nr, EMB), lambda i: (0, 0)),
            pl.BlockSpec((1, EMB), lambda i: (0, 0)),
            pl.BlockSpec((EMB, EMB), lambda i: (0, 0)),
            pl.BlockSpec((1, EMB), lambda i: (0, 0)),
        ],
        out_specs=pl.BlockSpec((blk, EMB), lambda i: (i, 0)),
        out_shape=jax.ShapeDtypeStruct((n_edges, EMB), jnp.float32),
    )(rbf, W_rbf, b_rbf.reshape(1, EMB), W[2 * EMB:3 * EMB, :],
      b.reshape(1, EMB))

    n_workers = 32
    per_worker = n_edges // n_workers
    chunk = 128
    while per_worker % chunk or chunk % 16:
        chunk -= 16
    n_chunks = per_worker // chunk

    mesh = plsc.VectorSubcoreMesh(core_axis_name="c", subcore_axis_name="s",
                                  num_cores=2, num_subcores=16)
    sc = pl.kernel(
        functools.partial(_sc_body, n_chunks, chunk),
        out_type=jax.ShapeDtypeStruct((n_edges, EMB), jnp.float32),
        mesh=mesh,
        scratch_types=[
            pltpu.VMEM((n_atoms,), jnp.int32),
            pltpu.VMEM((chunk,), jnp.int32),
            pltpu.VMEM((chunk,), jnp.int32),
            pltpu.VMEM((chunk,), jnp.int32),
            pltpu.VMEM((chunk,), jnp.int32),
            pltpu.VMEM((chunk, EMB), jnp.float32),
            pltpu.VMEM((chunk, EMB), jnp.float32),
            pltpu.VMEM((chunk, EMB), jnp.float32),
            pltpu.VMEM((chunk, EMB), jnp.float32),
            pltpu.SemaphoreType.DMA,
            pltpu.SemaphoreType.DMA,
        ],
        compiler_params=pltpu.CompilerParams(needs_layout_passes=False),
    )
    return sc(b1, b2, Z.astype(jnp.int32), idnb_i.astype(jnp.int32),
              idnb_j.astype(jnp.int32), y)


# speedup vs baseline: 2.8824x; 1.0668x over previous
import functools

import jax
import jax.numpy as jnp
from jax import lax
from jax.experimental import pallas as pl
from jax.experimental.pallas import tpu as pltpu
from jax.experimental.pallas import tpu_sc as plsc

EMB = 128
LANES = 16
COLS = EMB // LANES


def _tables_body(emb_ref, w_ref, b1_ref, b2_ref):
    e = emb_ref[...]
    b1_ref[...] = jnp.dot(e, w_ref[0:EMB, :], preferred_element_type=jnp.float32)
    b2_ref[...] = jnp.dot(e, w_ref[EMB:2 * EMB, :], preferred_element_type=jnp.float32)


def _y_body(rbf_ref, wr_ref, br_ref, w3_ref, b_ref, y_ref):
    t = jnp.dot(rbf_ref[...], wr_ref[...], preferred_element_type=jnp.float32)
    t = t + br_ref[...]
    s = t / (1.0 + jnp.exp(-t))
    y_ref[...] = jnp.dot(s, w3_ref[...], preferred_element_type=jnp.float32) + b_ref[...]


def _sc_body(n_chunks, chunk, b1_hbm, b2_hbm, z_hbm, ii_hbm, ij_hbm, y_hbm,
             out_hbm, z_v, ii_v, ij_v, zi_v, zj_v, g1b, g2b, yb, ob,
             si0, si1, sg0, sg1, so0, so1):
    wid = lax.axis_index("s") * 2 + lax.axis_index("c")
    base = wid * (n_chunks * chunk)
    si = (si0, si1)
    sg = (sg0, sg1)
    so = (so0, so1)
    last = n_chunks - 1

    pltpu.sync_copy(z_hbm, z_v)

    def _off(c):
        return base + jnp.minimum(c, last) * chunk

    def fire_idx(c, b):
        off = _off(c)
        pltpu.make_async_copy(ii_hbm.at[pl.ds(off, chunk)], ii_v.at[b],
                              si[b]).start()
        pltpu.make_async_copy(ij_hbm.at[pl.ds(off, chunk)], ij_v.at[b],
                              si[b]).start()

    def wait_idx(b):
        pltpu.make_async_copy(ii_hbm.at[pl.ds(0, chunk)], ii_v.at[b],
                              si[b]).wait()
        pltpu.make_async_copy(ij_hbm.at[pl.ds(0, chunk)], ij_v.at[b],
                              si[b]).wait()

    def fire_y(c, b):
        pltpu.make_async_copy(y_hbm.at[pl.ds(_off(c), chunk)], yb.at[b],
                              sg[b]).start()

    def index_and_fire_gathers(b):
        @plsc.parallel_loop(0, chunk // LANES)
        def index_body(g):
            sl = pl.ds(g * LANES, LANES)
            zi_v[b, sl] = plsc.load_gather(z_v, [ii_v[b, sl]])
            zj_v[b, sl] = plsc.load_gather(z_v, [ij_v[b, sl]])

        pltpu.make_async_copy(b1_hbm.at[zi_v.at[b]], g1b.at[b], sg[b]).start()
        pltpu.make_async_copy(b2_hbm.at[zj_v.at[b]], g2b.at[b], sg[b]).start()

    def wait_gy(b):
        pltpu.make_async_copy(y_hbm.at[pl.ds(0, chunk)], yb.at[b], sg[b]).wait()
        pltpu.make_async_copy(b1_hbm.at[zi_v.at[b]], g1b.at[b], sg[b]).wait()
        pltpu.make_async_copy(b2_hbm.at[zj_v.at[b]], g2b.at[b], sg[b]).wait()

    def compute(b):
        @plsc.parallel_loop(0, chunk, unroll=2)
        def row_body(r):
            for k in range(COLS):
                sl = pl.ds(k * LANES, LANES)
                acc = yb[b, r, sl] + g1b[b, r, sl] + g2b[b, r, sl]
                ob[b, r, sl] = acc / (1.0 + jnp.exp(-acc))

    def fire_out(c, b):
        pltpu.make_async_copy(ob.at[b], out_hbm.at[pl.ds(_off(c), chunk)],
                              so[b]).start()

    def wait_out(b):
        pltpu.make_async_copy(ob.at[b], out_hbm.at[pl.ds(0, chunk)],
                              so[b]).wait()

    fire_idx(0, 0)
    fire_y(0, 0)
    fire_idx(1, 1)
    fire_y(1, 1)
    wait_idx(0)
    index_and_fire_gathers(0)

    n_pairs = n_chunks // 2

    def pair_body(p, _):
        c = 2 * p
        fire_idx(c + 2, 0)
        wait_idx(1)
        index_and_fire_gathers(1)
        wait_gy(0)

        @pl.when(p > 0)
        def _():
            wait_out(0)

        compute(0)
        fire_out(c, 0)
        fire_y(c + 2, 0)
        wait_idx(0)
        index_and_fire_gathers(0)
        wait_gy(1)

        @pl.when(p > 0)
        def _():
            wait_out(1)

        compute(1)
        fire_out(c + 1, 1)
        fire_idx(c + 3, 1)
        fire_y(c + 3, 1)
        return 0

    lax.fori_loop(0, n_pairs, pair_body, 0)

    wait_gy(0)
    wait_out(0)
    compute(0)
    fire_out(last, 0)
    wait_idx(1)
    pltpu.make_async_copy(y_hbm.at[pl.ds(0, chunk)], yb.at[1], sg[1]).wait()
    wait_out(1)
    wait_out(0)


def kernel(Z, rbf, idnb_i, idnb_j, embeddings, W_rbf, b_rbf, W, b):
    n_edges = rbf.shape[0]
    n_atoms = Z.shape[0]
    nv = embeddings.shape[0]

    nv_pad = ((nv + 7) // 8) * 8
    emb_p = jnp.pad(embeddings, ((0, nv_pad - nv), (0, 0)))
    b1, b2 = pl.pallas_call(
        _tables_body,
        out_shape=(
            jax.ShapeDtypeStruct((nv_pad, EMB), jnp.float32),
            jax.ShapeDtypeStruct((nv_pad, EMB), jnp.float32),
        ),
    )(emb_p, W)

    blk = 2000
    grid = n_edges // blk
    nr = rbf.shape[1]
    y = pl.pallas_call(
        _y_body,
        grid=(grid,),
        in_specs=[
            pl.BlockSpec((blk, nr), lambda i: (i, 0)),
            pl.BlockSpec((nr, EMB), lambda i: (0, 0)),
            pl.BlockSpec((1, EMB), lambda i: (0, 0)),
            pl.BlockSpec((EMB, EMB), lambda i: (0, 0)),
            pl.BlockSpec((1, EMB), lambda i: (0, 0)),
        ],
        out_specs=pl.BlockSpec((blk, EMB), lambda i: (i, 0)),
        out_shape=jax.ShapeDtypeStruct((n_edges, EMB), jnp.float32),
    )(rbf, W_rbf, b_rbf.reshape(1, EMB), W[2 * EMB:3 * EMB, :],
      b.reshape(1, EMB))

    n_workers = 32
    per_worker = n_edges // n_workers
    chunk = 128
    while per_worker % chunk or chunk % 16:
        chunk -= 16
    n_chunks = per_worker // chunk

    mesh = plsc.VectorSubcoreMesh(core_axis_name="c", subcore_axis_name="s",
                                  num_cores=2, num_subcores=16)
    sc = pl.kernel(
        functools.partial(_sc_body, n_chunks, chunk),
        out_type=jax.ShapeDtypeStruct((n_edges, EMB), jnp.float32),
        mesh=mesh,
        scratch_types=[
            pltpu.VMEM((n_atoms,), jnp.int32),
            pltpu.VMEM((2, chunk), jnp.int32),
            pltpu.VMEM((2, chunk), jnp.int32),
            pltpu.VMEM((2, chunk), jnp.int32),
            pltpu.VMEM((2, chunk), jnp.int32),
            pltpu.VMEM((2, chunk, EMB), jnp.float32),
            pltpu.VMEM((2, chunk, EMB), jnp.float32),
            pltpu.VMEM((2, chunk, EMB), jnp.float32),
            pltpu.VMEM((2, chunk, EMB), jnp.float32),
            pltpu.SemaphoreType.DMA,
            pltpu.SemaphoreType.DMA,
            pltpu.SemaphoreType.DMA,
            pltpu.SemaphoreType.DMA,
            pltpu.SemaphoreType.DMA,
            pltpu.SemaphoreType.DMA,
        ],
        compiler_params=pltpu.CompilerParams(needs_layout_passes=False),
    )
    return sc(b1, b2, Z.astype(jnp.int32), idnb_i.astype(jnp.int32),
              idnb_j.astype(jnp.int32), y)


# speedup vs baseline: 4.7548x; 1.6496x over previous
import functools

import jax
import jax.numpy as jnp
from jax import lax
from jax.experimental import pallas as pl
from jax.experimental.pallas import tpu as pltpu
from jax.experimental.pallas import tpu_sc as plsc

EMB = 128
LANES = 16
COLS = EMB // LANES


def _tables_body(emb_ref, w_ref, b12_ref):
    e = emb_ref[...]
    b1 = jnp.dot(e, w_ref[0:EMB, :], preferred_element_type=jnp.float32)
    b2 = jnp.dot(e, w_ref[EMB:2 * EMB, :], preferred_element_type=jnp.float32)
    b12_ref[...] = b1[:, None, :] + b2[None, :, :]


def _y_body(rbf_ref, wr_ref, br_ref, w3_ref, b_ref, y_ref):
    t = jnp.dot(rbf_ref[...], wr_ref[...], preferred_element_type=jnp.float32)
    t = t + br_ref[...]
    s = t / (1.0 + jnp.exp(-t))
    y_ref[...] = jnp.dot(s, w3_ref[...], preferred_element_type=jnp.float32) + b_ref[...]


def _sc_body(n_chunks, chunk, nv_pad, b12_hbm, z_hbm, ii_hbm, ij_hbm, y_hbm,
             out_hbm, z_v, ii_v, ij_v, zp_v, gb, yb, ob,
             si0, si1, sg0, sg1, so0, so1):
    wid = lax.axis_index("s") * 2 + lax.axis_index("c")
    base = wid * (n_chunks * chunk)
    si = (si0, si1)
    sg = (sg0, sg1)
    so = (so0, so1)
    last = n_chunks - 1

    pltpu.sync_copy(z_hbm, z_v)

    def _off(c):
        return base + jnp.minimum(c, last) * chunk

    def fire_idx(c, b):
        off = _off(c)
        pltpu.make_async_copy(ii_hbm.at[pl.ds(off, chunk)], ii_v.at[b],
                              si[b]).start()
        pltpu.make_async_copy(ij_hbm.at[pl.ds(off, chunk)], ij_v.at[b],
                              si[b]).start()

    def wait_idx(b):
        pltpu.make_async_copy(ii_hbm.at[pl.ds(0, chunk)], ii_v.at[b],
                              si[b]).wait()
        pltpu.make_async_copy(ij_hbm.at[pl.ds(0, chunk)], ij_v.at[b],
                              si[b]).wait()

    def fire_y(c, b):
        pltpu.make_async_copy(y_hbm.at[pl.ds(_off(c), chunk)], yb.at[b],
                              sg[b]).start()

    def index_and_fire_gathers(b):
        @plsc.parallel_loop(0, chunk // LANES)
        def index_body(g):
            sl = pl.ds(g * LANES, LANES)
            zi = plsc.load_gather(z_v, [ii_v[b, sl]])
            zj = plsc.load_gather(z_v, [ij_v[b, sl]])
            zp_v[b, sl] = zi * nv_pad + zj

        pltpu.make_async_copy(b12_hbm.at[zp_v.at[b]], gb.at[b], sg[b]).start()

    def wait_gy(b):
        pltpu.make_async_copy(y_hbm.at[pl.ds(0, chunk)], yb.at[b], sg[b]).wait()
        pltpu.make_async_copy(b12_hbm.at[zp_v.at[b]], gb.at[b], sg[b]).wait()

    def compute(b):
        @plsc.parallel_loop(0, chunk, unroll=2)
        def row_body(r):
            for k in range(COLS):
                sl = pl.ds(k * LANES, LANES)
                acc = yb[b, r, sl] + gb[b, r, sl]
                ob[b, r, sl] = acc / (1.0 + jnp.exp(-acc))

    def fire_out(c, b):
        pltpu.make_async_copy(ob.at[b], out_hbm.at[pl.ds(_off(c), chunk)],
                              so[b]).start()

    def wait_out(b):
        pltpu.make_async_copy(ob.at[b], out_hbm.at[pl.ds(0, chunk)],
                              so[b]).wait()

    fire_idx(0, 0)
    fire_y(0, 0)
    fire_idx(1, 1)
    fire_y(1, 1)
    wait_idx(0)
    index_and_fire_gathers(0)

    n_pairs = n_chunks // 2

    def pair_body(p, _):
        c = 2 * p
        fire_idx(c + 2, 0)
        wait_idx(1)
        index_and_fire_gathers(1)
        wait_gy(0)

        @pl.when(p > 0)
        def _():
            wait_out(0)

        compute(0)
        fire_out(c, 0)
        fire_y(c + 2, 0)
        wait_idx(0)
        index_and_fire_gathers(0)
        wait_gy(1)

        @pl.when(p > 0)
        def _():
            wait_out(1)

        compute(1)
        fire_out(c + 1, 1)
        fire_idx(c + 3, 1)
        fire_y(c + 3, 1)
        return 0

    lax.fori_loop(0, n_pairs, pair_body, 0)

    wait_gy(0)
    wait_out(0)
    compute(0)
    fire_out(last, 0)
    wait_idx(1)
    pltpu.make_async_copy(y_hbm.at[pl.ds(0, chunk)], yb.at[1], sg[1]).wait()
    wait_out(1)
    wait_out(0)


def kernel(Z, rbf, idnb_i, idnb_j, embeddings, W_rbf, b_rbf, W, b):
    n_edges = rbf.shape[0]
    n_atoms = Z.shape[0]
    nv = embeddings.shape[0]

    nv_pad = ((nv + 7) // 8) * 8
    emb_p = jnp.pad(embeddings, ((0, nv_pad - nv), (0, 0)))
    b12 = pl.pallas_call(
        _tables_body,
        out_shape=jax.ShapeDtypeStruct((nv_pad, nv_pad, EMB), jnp.float32),
    )(emb_p, W)
    b12 = b12.reshape(nv_pad * nv_pad, EMB)

    blk = 2000
    grid = n_edges // blk
    nr = rbf.shape[1]
    y = pl.pallas_call(
        _y_body,
        grid=(grid,),
        in_specs=[
            pl.BlockSpec((blk, nr), lambda i: (i, 0)),
            pl.BlockSpec((nr, EMB), lambda i: (0, 0)),
            pl.BlockSpec((1, EMB), lambda i: (0, 0)),
            pl.BlockSpec((EMB, EMB), lambda i: (0, 0)),
            pl.BlockSpec((1, EMB), lambda i: (0, 0)),
        ],
        out_specs=pl.BlockSpec((blk, EMB), lambda i: (i, 0)),
        out_shape=jax.ShapeDtypeStruct((n_edges, EMB), jnp.float32),
    )(rbf, W_rbf, b_rbf.reshape(1, EMB), W[2 * EMB:3 * EMB, :],
      b.reshape(1, EMB))

    n_workers = 32
    per_worker = n_edges // n_workers
    chunk = 128
    while per_worker % chunk or chunk % 16:
        chunk -= 16
    n_chunks = per_worker // chunk

    mesh = plsc.VectorSubcoreMesh(core_axis_name="c", subcore_axis_name="s",
                                  num_cores=2, num_subcores=16)
    sc = pl.kernel(
        functools.partial(_sc_body, n_chunks, chunk, nv_pad),
        out_type=jax.ShapeDtypeStruct((n_edges, EMB), jnp.float32),
        mesh=mesh,
        scratch_types=[
            pltpu.VMEM((n_atoms,), jnp.int32),
            pltpu.VMEM((2, chunk), jnp.int32),
            pltpu.VMEM((2, chunk), jnp.int32),
            pltpu.VMEM((2, chunk), jnp.int32),
            pltpu.VMEM((2, chunk, EMB), jnp.float32),
            pltpu.VMEM((2, chunk, EMB), jnp.float32),
            pltpu.VMEM((2, chunk, EMB), jnp.float32),
            pltpu.SemaphoreType.DMA,
            pltpu.SemaphoreType.DMA,
            pltpu.SemaphoreType.DMA,
            pltpu.SemaphoreType.DMA,
            pltpu.SemaphoreType.DMA,
            pltpu.SemaphoreType.DMA,
        ],
        compiler_params=pltpu.CompilerParams(needs_layout_passes=False),
    )
    return sc(b12, Z.astype(jnp.int32), idnb_i.astype(jnp.int32),
              idnb_j.astype(jnp.int32), y)


# speedup vs baseline: 5.0763x; 1.0676x over previous
import functools

import jax
import jax.numpy as jnp
from jax import lax
from jax.experimental import pallas as pl
from jax.experimental.pallas import tpu as pltpu
from jax.experimental.pallas import tpu_sc as plsc

EMB = 128
LANES = 16
COLS = EMB // LANES

def _swish(x):
    return x / (1.0 + jnp.exp(-x))


def _pack_cols(y):
    u = lax.bitcast_convert_type(y.astype(jnp.bfloat16), jnp.uint16)
    u = u.astype(jnp.uint32)
    lo = jnp.concatenate([u[..., 32 * t:32 * t + 16] for t in range(4)],
                         axis=-1)
    hi = jnp.concatenate([u[..., 32 * t + 16:32 * t + 32] for t in range(4)],
                         axis=-1)
    return lax.bitcast_convert_type(lo | (hi << 16), jnp.float32)


def _tables_body(emb_ref, wp_ref, b12_ref):
    e = emb_ref[...]
    b1 = jnp.dot(e, wp_ref[0:EMB, :], preferred_element_type=jnp.float32)
    b2 = jnp.dot(e, wp_ref[EMB:2 * EMB, :], preferred_element_type=jnp.float32)
    b12_ref[...] = b1[:, None, :] + b2[None, :, :]


def _y_body(rbft_ref, wr_ref, br_ref, w3_ref, b_ref, y2_ref):
    rt = rbft_ref[...]
    t = lax.dot_general(rt, wr_ref[...], (((0,), (0,)), ((), ())),
                        preferred_element_type=jnp.float32)
    t = t + br_ref[...]
    s = _swish(t)
    y = jnp.dot(s, w3_ref[...], preferred_element_type=jnp.float32)
    y = y + b_ref[...]
    u = lax.bitcast_convert_type(y.astype(jnp.bfloat16), jnp.uint16)
    u = u.astype(jnp.uint32).reshape(y.shape[0] // 2, 2, EMB)
    p = u[:, 0, :] | (u[:, 1, :] << 16)
    y2_ref[...] = lax.bitcast_convert_type(p, jnp.float32)


def _sc_body(n_chunks, chunk, nv_pad, b12_hbm, z_hbm, ii_hbm, ij_hbm, y2_hbm,
             out_hbm, z_v, ii_v, ij_v, zp_v, gb, yb, ob,
             si0, si1, sg0, sg1, so0, so1):
    wid = lax.axis_index("s") * 2 + lax.axis_index("c")
    base = wid * (n_chunks * chunk)
    si = (si0, si1)
    sg = (sg0, sg1)
    so = (so0, so1)
    last = n_chunks - 1
    half = chunk // 2

    pltpu.sync_copy(z_hbm, z_v)

    def _off(c):
        return pl.multiple_of(base + jnp.minimum(c, last) * chunk, chunk)

    def fire_idx(c, b):
        off = _off(c)
        pltpu.make_async_copy(ii_hbm.at[pl.ds(off, chunk)], ii_v.at[b],
                              si[b]).start()
        pltpu.make_async_copy(ij_hbm.at[pl.ds(off, chunk)], ij_v.at[b],
                              si[b]).start()

    def wait_idx(b):
        pltpu.make_async_copy(ii_hbm.at[pl.ds(0, chunk)], ii_v.at[b],
                              si[b]).wait()
        pltpu.make_async_copy(ij_hbm.at[pl.ds(0, chunk)], ij_v.at[b],
                              si[b]).wait()

    def fire_y(c, b):
        off2 = pl.multiple_of(_off(c) // 2, half)
        pltpu.make_async_copy(y2_hbm.at[pl.ds(off2, half)], yb.at[b],
                              sg[b]).start()

    def _z_lookup(idx):
        zr = plsc.load_gather(z_v, [lax.shift_right_logical(idx, 1)])
        odd = lax.bitwise_and(idx, 1)
        return jnp.where(odd == 1, lax.shift_right_logical(zr, 16),
                         lax.bitwise_and(zr, 0xFFFF))

    def index_and_fire_gathers(b):
        @plsc.parallel_loop(0, chunk // LANES)
        def index_body(g):
            sl = pl.ds(g * LANES, LANES)
            zi = _z_lookup(ii_v[b, sl])
            zj = _z_lookup(ij_v[b, sl])
            zp_v[b, sl] = zi * nv_pad + zj

        pltpu.make_async_copy(b12_hbm.at[zp_v.at[b]], gb.at[b], sg[b]).start()

    def wait_gy(b):
        pltpu.make_async_copy(y2_hbm.at[pl.ds(0, half)], yb.at[b],
                              sg[b]).wait()
        pltpu.make_async_copy(b12_hbm.at[zp_v.at[b]], gb.at[b], sg[b]).wait()

    def compute(b):
        @plsc.parallel_loop(0, half)
        def row_body(r):
            e0 = 2 * r
            e1 = 2 * r + 1
            for q in range(COLS):
                sl = pl.ds(q * LANES, LANES)
                y0, y1 = plsc.unpack(
                    plsc.bitcast(yb[b, r, sl], jnp.bfloat16),
                    format=plsc.PackFormat.INTERLEAVED,
                    preferred_element_type=jnp.float32)
                ob[b, e0, sl] = _swish(gb[b, e0, sl] + y0)
                ob[b, e1, sl] = _swish(gb[b, e1, sl] + y1)

    def fire_out(c, b):
        pltpu.make_async_copy(ob.at[b], out_hbm.at[pl.ds(_off(c), chunk)],
                              so[b]).start()

    def wait_out(b):
        pltpu.make_async_copy(ob.at[b], out_hbm.at[pl.ds(0, chunk)],
                              so[b]).wait()

    fire_idx(0, 0)
    fire_y(0, 0)
    fire_idx(1, 1)
    fire_y(1, 1)
    wait_idx(0)
    index_and_fire_gathers(0)

    n_pairs = n_chunks // 2

    def pair_body(p, _):
        c = 2 * p
        fire_idx(c + 2, 0)
        wait_idx(1)
        index_and_fire_gathers(1)
        wait_gy(0)

        @pl.when(p > 0)
        def _():
            wait_out(0)

        compute(0)
        fire_out(c, 0)
        fire_y(c + 2, 0)
        wait_idx(0)
        index_and_fire_gathers(0)
        wait_gy(1)

        @pl.when(p > 0)
        def _():
            wait_out(1)

        compute(1)
        fire_out(c + 1, 1)
        fire_idx(c + 3, 1)
        fire_y(c + 3, 1)
        return 0

    lax.fori_loop(0, n_pairs, pair_body, 0)

    wait_gy(0)
    wait_out(0)
    compute(0)
    fire_out(last, 0)
    wait_idx(1)
    pltpu.make_async_copy(y2_hbm.at[pl.ds(0, half)], yb.at[1], sg[1]).wait()
    wait_out(1)
    wait_out(0)


def kernel(Z, rbf, idnb_i, idnb_j, embeddings, W_rbf, b_rbf, W, b):
    n_edges = rbf.shape[0]
    n_atoms = Z.shape[0]
    nv = embeddings.shape[0]
    nr = rbf.shape[1]
    nv_pad = ((nv + 7) // 8) * 8
    emb_p = jnp.pad(embeddings, ((0, nv_pad - nv), (0, 0)))
    b12 = pl.pallas_call(
        _tables_body,
        out_shape=jax.ShapeDtypeStruct((nv_pad, nv_pad, EMB), jnp.float32),
    )(emb_p, W[0:2 * EMB])
    b12 = b12.reshape(nv_pad * nv_pad, EMB)

    blk = 2560
    grid = n_edges // blk
    rbf_t = rbf.T
    y2 = pl.pallas_call(
        _y_body,
        grid=(grid,),
        in_specs=[
            pl.BlockSpec((nr, blk), lambda i: (0, i)),
            pl.BlockSpec((nr, EMB), lambda i: (0, 0)),
            pl.BlockSpec((1, EMB), lambda i: (0, 0)),
            pl.BlockSpec((EMB, EMB), lambda i: (0, 0)),
            pl.BlockSpec((1, EMB), lambda i: (0, 0)),
        ],
        out_specs=pl.BlockSpec((blk // 2, EMB), lambda i: (i, 0)),
        out_shape=jax.ShapeDtypeStruct((n_edges // 2, EMB), jnp.float32),
    )(rbf_t, W_rbf, b_rbf.reshape(1, EMB), W[2 * EMB:3 * EMB, :],
      b.reshape(1, EMB))

    n_workers = 32
    per_worker = n_edges // n_workers
    chunk = 128
    while per_worker % chunk or chunk % 16:
        chunk -= 16
    n_chunks = per_worker // chunk

    mesh = plsc.VectorSubcoreMesh(core_axis_name="c", subcore_axis_name="s",
                                  num_cores=2, num_subcores=16)
    sc = pl.kernel(
        functools.partial(_sc_body, n_chunks, chunk, nv_pad),
        out_type=jax.ShapeDtypeStruct((n_edges, EMB), jnp.float32),
        mesh=mesh,
        scratch_types=[
            pltpu.VMEM((n_atoms // 2,), jnp.int32),
            pltpu.VMEM((2, chunk), jnp.int32),
            pltpu.VMEM((2, chunk), jnp.int32),
            pltpu.VMEM((2, chunk), jnp.int32),
            pltpu.VMEM((2, chunk, EMB), jnp.float32),
            pltpu.VMEM((2, chunk // 2, EMB), jnp.float32),
            pltpu.VMEM((2, chunk, EMB), jnp.float32),
            pltpu.SemaphoreType.DMA,
            pltpu.SemaphoreType.DMA,
            pltpu.SemaphoreType.DMA,
            pltpu.SemaphoreType.DMA,
            pltpu.SemaphoreType.DMA,
            pltpu.SemaphoreType.DMA,
        ],
        compiler_params=pltpu.CompilerParams(needs_layout_passes=False),
    )
    z32 = Z.astype(jnp.int32)
    z_pk = z32[0::2] | (z32[1::2] << 16)
    return sc(b12, z_pk, idnb_i.astype(jnp.int32),
              idnb_j.astype(jnp.int32), y2)


# speedup vs baseline: 6.4198x; 1.2647x over previous
import functools

import jax
import jax.numpy as jnp
from jax import lax
from jax.experimental import pallas as pl
from jax.experimental.pallas import tpu as pltpu
from jax.experimental.pallas import tpu_sc as plsc

EMB = 128
LANES = 16
COLS = EMB // LANES

def _swish(x):
    return x / (1.0 + jnp.exp(-x))


def _pack_cols(y):
    u = lax.bitcast_convert_type(y.astype(jnp.bfloat16), jnp.uint16)
    u = u.astype(jnp.uint32)
    lo = jnp.concatenate([u[..., 32 * t:32 * t + 16] for t in range(4)],
                         axis=-1)
    hi = jnp.concatenate([u[..., 32 * t + 16:32 * t + 32] for t in range(4)],
                         axis=-1)
    return lax.bitcast_convert_type(lo | (hi << 16), jnp.float32)


def _tables_body(emb_ref, wp_ref, b12_ref):
    e = emb_ref[...]
    b1 = jnp.dot(e, wp_ref[0:EMB, :], preferred_element_type=jnp.float32)
    b2 = jnp.dot(e, wp_ref[EMB:2 * EMB, :], preferred_element_type=jnp.float32)
    b12_ref[...] = b1[:, None, :] + b2[None, :, :]


def _y_body(rbft_ref, wr_ref, br_ref, w3_ref, b_ref, y2_ref):
    rt = rbft_ref[...]
    t = lax.dot_general(rt, wr_ref[...], (((0,), (0,)), ((), ())),
                        preferred_element_type=jnp.float32)
    t = t + br_ref[...]
    s = _swish(t)
    y = jnp.dot(s, w3_ref[...], preferred_element_type=jnp.float32)
    y = y + b_ref[...]
    y2_ref[...] = pltpu.bitcast(y.astype(jnp.bfloat16), jnp.float32)


def _sc_body(n_chunks, chunk, nv_pad, b12_hbm, z_hbm, ii_hbm, ij_hbm, y2_hbm,
             out_hbm, z_v, ii_v, ij_v, zp_v, gb, yb, ob,
             si0, si1, sg0, sg1, so0, so1):
    wid = lax.axis_index("s") * 2 + lax.axis_index("c")
    base = wid * (n_chunks * chunk)
    si = (si0, si1)
    sg = (sg0, sg1)
    so = (so0, so1)
    last = n_chunks - 1
    half = chunk // 2

    pltpu.sync_copy(z_hbm, z_v)

    def _off(c):
        return pl.multiple_of(base + jnp.minimum(c, last) * chunk, chunk)

    def fire_idx(c, b):
        off = _off(c)
        pltpu.make_async_copy(ii_hbm.at[pl.ds(off, chunk)], ii_v.at[b],
                              si[b]).start()
        pltpu.make_async_copy(ij_hbm.at[pl.ds(off, chunk)], ij_v.at[b],
                              si[b]).start()

    def wait_idx(b):
        pltpu.make_async_copy(ii_hbm.at[pl.ds(0, chunk)], ii_v.at[b],
                              si[b]).wait()
        pltpu.make_async_copy(ij_hbm.at[pl.ds(0, chunk)], ij_v.at[b],
                              si[b]).wait()

    def fire_y(c, b):
        off2 = pl.multiple_of(_off(c) // 2, half)
        pltpu.make_async_copy(y2_hbm.at[pl.ds(off2, half)], yb.at[b],
                              sg[b]).start()

    def _z_lookup(idx):
        zr = plsc.load_gather(z_v, [lax.shift_right_logical(idx, 1)])
        odd = lax.bitwise_and(idx, 1)
        return jnp.where(odd == 1, lax.shift_right_logical(zr, 16),
                         lax.bitwise_and(zr, 0xFFFF))

    def index_and_fire_gathers(b):
        @plsc.parallel_loop(0, chunk // LANES)
        def index_body(g):
            sl = pl.ds(g * LANES, LANES)
            zi = _z_lookup(ii_v[b, sl])
            zj = _z_lookup(ij_v[b, sl])
            zp_v[b, sl] = zi * nv_pad + zj

        pltpu.make_async_copy(b12_hbm.at[zp_v.at[b]], gb.at[b], sg[b]).start()

    def wait_gy(b):
        pltpu.make_async_copy(y2_hbm.at[pl.ds(0, half)], yb.at[b],
                              sg[b]).wait()
        pltpu.make_async_copy(b12_hbm.at[zp_v.at[b]], gb.at[b], sg[b]).wait()

    def compute(b):
        @plsc.parallel_loop(0, half)
        def row_body(r):
            e0 = 2 * r
            e1 = e0 + 1
            for q in range(COLS):
                sl = pl.ds(q * LANES, LANES)
                y0, y1 = plsc.unpack(
                    plsc.bitcast(yb[b, r, sl], jnp.bfloat16),
                    format=plsc.PackFormat.INTERLEAVED,
                    preferred_element_type=jnp.float32)
                ob[b, e0, sl] = _swish(gb[b, e0, sl] + y0)
                ob[b, e1, sl] = _swish(gb[b, e1, sl] + y1)

    def fire_out(c, b):
        pltpu.make_async_copy(ob.at[b], out_hbm.at[pl.ds(_off(c), chunk)],
                              so[b]).start()

    def wait_out(b):
        pltpu.make_async_copy(ob.at[b], out_hbm.at[pl.ds(0, chunk)],
                              so[b]).wait()

    fire_idx(0, 0)
    fire_y(0, 0)
    fire_idx(1, 1)
    fire_y(1, 1)
    wait_idx(0)
    index_and_fire_gathers(0)

    n_pairs = n_chunks // 2

    def pair_body(p, _):
        c = 2 * p
        fire_idx(c + 2, 0)
        wait_idx(1)
        index_and_fire_gathers(1)
        wait_gy(0)

        @pl.when(p > 0)
        def _():
            wait_out(0)

        compute(0)
        fire_out(c, 0)
        fire_y(c + 2, 0)
        wait_idx(0)
        index_and_fire_gathers(0)
        wait_gy(1)

        @pl.when(p > 0)
        def _():
            wait_out(1)

        compute(1)
        fire_out(c + 1, 1)
        fire_idx(c + 3, 1)
        fire_y(c + 3, 1)
        return 0

    lax.fori_loop(0, n_pairs, pair_body, 0)

    wait_gy(0)
    wait_out(0)
    compute(0)
    fire_out(last, 0)
    wait_idx(1)
    pltpu.make_async_copy(y2_hbm.at[pl.ds(0, half)], yb.at[1], sg[1]).wait()
    wait_out(1)
    wait_out(0)


def kernel(Z, rbf, idnb_i, idnb_j, embeddings, W_rbf, b_rbf, W, b):
    n_edges = rbf.shape[0]
    n_atoms = Z.shape[0]
    nv = embeddings.shape[0]
    nr = rbf.shape[1]
    nv_pad = ((nv + 7) // 8) * 8
    emb_p = jnp.pad(embeddings, ((0, nv_pad - nv), (0, 0)))
    b12 = pl.pallas_call(
        _tables_body,
        out_shape=jax.ShapeDtypeStruct((nv_pad, nv_pad, EMB), jnp.float32),
    )(emb_p, W[0:2 * EMB])
    b12 = b12.reshape(nv_pad * nv_pad, EMB)

    blk = 2560
    grid = n_edges // blk
    rbf_t = rbf.T
    y2 = pl.pallas_call(
        _y_body,
        grid=(grid,),
        in_specs=[
            pl.BlockSpec((nr, blk), lambda i: (0, i)),
            pl.BlockSpec((nr, EMB), lambda i: (0, 0)),
            pl.BlockSpec((1, EMB), lambda i: (0, 0)),
            pl.BlockSpec((EMB, EMB), lambda i: (0, 0)),
            pl.BlockSpec((1, EMB), lambda i: (0, 0)),
        ],
        out_specs=pl.BlockSpec((blk // 2, EMB), lambda i: (i, 0)),
        out_shape=jax.ShapeDtypeStruct((n_edges // 2, EMB), jnp.float32),
    )(rbf_t, W_rbf, b_rbf.reshape(1, EMB), W[2 * EMB:3 * EMB, :],
      b.reshape(1, EMB))

    n_workers = 32
    per_worker = n_edges // n_workers
    chunk = 128
    while per_worker % chunk or chunk % 16:
        chunk -= 16
    n_chunks = per_worker // chunk

    mesh = plsc.VectorSubcoreMesh(core_axis_name="c", subcore_axis_name="s",
                                  num_cores=2, num_subcores=16)
    sc = pl.kernel(
        functools.partial(_sc_body, n_chunks, chunk, nv_pad),
        out_type=jax.ShapeDtypeStruct((n_edges, EMB), jnp.float32),
        mesh=mesh,
        scratch_types=[
            pltpu.VMEM((n_atoms // 2,), jnp.int32),
            pltpu.VMEM((2, chunk), jnp.int32),
            pltpu.VMEM((2, chunk), jnp.int32),
            pltpu.VMEM((2, chunk), jnp.int32),
            pltpu.VMEM((2, chunk, EMB), jnp.float32),
            pltpu.VMEM((2, chunk // 2, EMB), jnp.float32),
            pltpu.VMEM((2, chunk, EMB), jnp.float32),
            pltpu.SemaphoreType.DMA,
            pltpu.SemaphoreType.DMA,
            pltpu.SemaphoreType.DMA,
            pltpu.SemaphoreType.DMA,
            pltpu.SemaphoreType.DMA,
            pltpu.SemaphoreType.DMA,
        ],
        compiler_params=pltpu.CompilerParams(needs_layout_passes=False),
    )
    z32 = Z.astype(jnp.int32)
    z_pk = z32[0::2] | (z32[1::2] << 16)
    return sc(b12, z_pk, idnb_i.astype(jnp.int32),
              idnb_j.astype(jnp.int32), y2)


# speedup vs baseline: 7.1677x; 1.1165x over previous
import functools

import jax
import jax.numpy as jnp
from jax import lax
from jax.experimental import pallas as pl
from jax.experimental.pallas import tpu as pltpu
from jax.experimental.pallas import tpu_sc as plsc

EMB = 128
LANES = 16
COLS = EMB // LANES

def _swish(x):
    return x / (1.0 + jnp.exp(-x))


def _pack_cols(y):
    u = lax.bitcast_convert_type(y.astype(jnp.bfloat16), jnp.uint16)
    u = u.astype(jnp.uint32)
    lo = jnp.concatenate([u[..., 32 * t:32 * t + 16] for t in range(4)],
                         axis=-1)
    hi = jnp.concatenate([u[..., 32 * t + 16:32 * t + 32] for t in range(4)],
                         axis=-1)
    return lax.bitcast_convert_type(lo | (hi << 16), jnp.float32)


def _tables_body(emb_ref, wp_ref, b12_ref):
    e = emb_ref[...]
    b1 = jnp.dot(e, wp_ref[0:EMB, :], preferred_element_type=jnp.float32)
    b2 = jnp.dot(e, wp_ref[EMB:2 * EMB, :], preferred_element_type=jnp.float32)
    b12_ref[...] = b1[:, None, :] + b2[None, :, :]


def _y_body(rbft_ref, wr_ref, br_ref, w3_ref, b_ref, y2_ref):
    rt = rbft_ref[...]
    t = lax.dot_general(rt, wr_ref[...], (((0,), (0,)), ((), ())),
                        preferred_element_type=jnp.float32)
    t = t + br_ref[...]
    s = jax.nn.swish(t).astype(jnp.bfloat16)
    y = jnp.dot(s, w3_ref[...], preferred_element_type=jnp.float32)
    y = y + b_ref[...]
    y2_ref[...] = pltpu.bitcast(y.astype(jnp.bfloat16), jnp.float32)


def _sc_body(n_chunks, chunk, nv_pad, b12_hbm, z_hbm, ii_hbm, ij_hbm, y2_hbm,
             out_hbm, z_v, ii_v, ij_v, zp_v, gb, yb, ob,
             si0, si1, sg0, sg1, so0, so1):
    wid = lax.axis_index("s") * 2 + lax.axis_index("c")
    base = wid * (n_chunks * chunk)
    si = (si0, si1)
    sg = (sg0, sg1)
    so = (so0, so1)
    last = n_chunks - 1
    half = chunk // 2

    pltpu.sync_copy(z_hbm, z_v)

    def _off(c):
        return pl.multiple_of(base + jnp.minimum(c, last) * chunk, chunk)

    def fire_idx(c, b):
        off = _off(c)
        pltpu.make_async_copy(ii_hbm.at[pl.ds(off, chunk)], ii_v.at[b],
                              si[b]).start()
        pltpu.make_async_copy(ij_hbm.at[pl.ds(off, chunk)], ij_v.at[b],
                              si[b]).start()

    def wait_idx(b):
        pltpu.make_async_copy(ii_hbm.at[pl.ds(0, chunk)], ii_v.at[b],
                              si[b]).wait()
        pltpu.make_async_copy(ij_hbm.at[pl.ds(0, chunk)], ij_v.at[b],
                              si[b]).wait()

    def fire_y(c, b):
        off2 = pl.multiple_of(_off(c) // 2, half)
        pltpu.make_async_copy(y2_hbm.at[pl.ds(off2, half)], yb.at[b],
                              sg[b]).start()

    def _z_lookup(idx):
        zr = plsc.load_gather(z_v, [lax.shift_right_logical(idx, 1)])
        odd = lax.bitwise_and(idx, 1)
        return jnp.where(odd == 1, lax.shift_right_logical(zr, 16),
                         lax.bitwise_and(zr, 0xFFFF))

    def index_and_fire_gathers(b):
        @plsc.parallel_loop(0, chunk // LANES)
        def index_body(g):
            sl = pl.ds(g * LANES, LANES)
            zi = _z_lookup(ii_v[b, sl])
            zj = _z_lookup(ij_v[b, sl])
            zp_v[b, sl] = zi * nv_pad + zj

        pltpu.make_async_copy(b12_hbm.at[zp_v.at[b]], gb.at[b], sg[b]).start()

    def wait_gy(b):
        pltpu.make_async_copy(y2_hbm.at[pl.ds(0, half)], yb.at[b],
                              sg[b]).wait()
        pltpu.make_async_copy(b12_hbm.at[zp_v.at[b]], gb.at[b], sg[b]).wait()

    def compute(b):
        @plsc.parallel_loop(0, half)
        def row_body(r):
            e0 = 2 * r
            e1 = e0 + 1
            for q in range(COLS):
                sl = pl.ds(q * LANES, LANES)
                y0, y1 = plsc.unpack(
                    plsc.bitcast(yb[b, r, sl], jnp.bfloat16),
                    format=plsc.PackFormat.INTERLEAVED,
                    preferred_element_type=jnp.float32)
                ob[b, e0, sl] = _swish(gb[b, e0, sl] + y0)
                ob[b, e1, sl] = _swish(gb[b, e1, sl] + y1)

    def fire_out(c, b):
        pltpu.make_async_copy(ob.at[b], out_hbm.at[pl.ds(_off(c), chunk)],
                              so[b]).start()

    def wait_out(b):
        pltpu.make_async_copy(ob.at[b], out_hbm.at[pl.ds(0, chunk)],
                              so[b]).wait()

    fire_idx(0, 0)
    fire_y(0, 0)
    fire_idx(1, 1)
    fire_y(1, 1)
    wait_idx(0)
    index_and_fire_gathers(0)

    n_pairs = n_chunks // 2

    def pair_body(p, _):
        c = 2 * p
        fire_idx(c + 2, 0)
        wait_idx(1)
        index_and_fire_gathers(1)
        wait_gy(0)

        @pl.when(p > 0)
        def _():
            wait_out(0)

        compute(0)
        fire_out(c, 0)
        fire_y(c + 2, 0)
        wait_idx(0)
        index_and_fire_gathers(0)
        wait_gy(1)

        @pl.when(p > 0)
        def _():
            wait_out(1)

        compute(1)
        fire_out(c + 1, 1)
        fire_idx(c + 3, 1)
        fire_y(c + 3, 1)
        return 0

    lax.fori_loop(0, n_pairs, pair_body, 0)

    wait_gy(0)
    wait_out(0)
    compute(0)
    fire_out(last, 0)
    wait_idx(1)
    pltpu.make_async_copy(y2_hbm.at[pl.ds(0, half)], yb.at[1], sg[1]).wait()
    wait_out(1)
    wait_out(0)


def kernel(Z, rbf, idnb_i, idnb_j, embeddings, W_rbf, b_rbf, W, b):
    n_edges = rbf.shape[0]
    n_atoms = Z.shape[0]
    nv = embeddings.shape[0]
    nr = rbf.shape[1]
    nv_pad = ((nv + 7) // 8) * 8
    emb_p = jnp.pad(embeddings, ((0, nv_pad - nv), (0, 0)))
    b12 = pl.pallas_call(
        _tables_body,
        out_shape=jax.ShapeDtypeStruct((nv_pad, nv_pad, EMB), jnp.float32),
    )(emb_p, W[0:2 * EMB])
    b12 = b12.reshape(nv_pad * nv_pad, EMB)

    blk = 6400
    grid = n_edges // blk
    rbf_t = rbf.T
    y2 = pl.pallas_call(
        _y_body,
        grid=(grid,),
        in_specs=[
            pl.BlockSpec((nr, blk), lambda i: (0, i)),
            pl.BlockSpec((nr, EMB), lambda i: (0, 0)),
            pl.BlockSpec((1, EMB), lambda i: (0, 0)),
            pl.BlockSpec((EMB, EMB), lambda i: (0, 0)),
            pl.BlockSpec((1, EMB), lambda i: (0, 0)),
        ],
        out_specs=pl.BlockSpec((blk // 2, EMB), lambda i: (i, 0)),
        out_shape=jax.ShapeDtypeStruct((n_edges // 2, EMB), jnp.float32),
    )(rbf_t, W_rbf, b_rbf.reshape(1, EMB),
      W[2 * EMB:3 * EMB, :].astype(jnp.bfloat16), b.reshape(1, EMB))

    n_workers = 32
    per_worker = n_edges // n_workers
    chunk = 128
    while per_worker % chunk or chunk % 16:
        chunk -= 16
    n_chunks = per_worker // chunk

    mesh = plsc.VectorSubcoreMesh(core_axis_name="c", subcore_axis_name="s",
                                  num_cores=2, num_subcores=16)
    sc = pl.kernel(
        functools.partial(_sc_body, n_chunks, chunk, nv_pad),
        out_type=jax.ShapeDtypeStruct((n_edges, EMB), jnp.float32),
        mesh=mesh,
        scratch_types=[
            pltpu.VMEM((n_atoms // 2,), jnp.int32),
            pltpu.VMEM((2, chunk), jnp.int32),
            pltpu.VMEM((2, chunk), jnp.int32),
            pltpu.VMEM((2, chunk), jnp.int32),
            pltpu.VMEM((2, chunk, EMB), jnp.float32),
            pltpu.VMEM((2, chunk // 2, EMB), jnp.float32),
            pltpu.VMEM((2, chunk, EMB), jnp.float32),
            pltpu.SemaphoreType.DMA,
            pltpu.SemaphoreType.DMA,
            pltpu.SemaphoreType.DMA,
            pltpu.SemaphoreType.DMA,
            pltpu.SemaphoreType.DMA,
            pltpu.SemaphoreType.DMA,
        ],
        compiler_params=pltpu.CompilerParams(needs_layout_passes=False),
    )
    z32 = Z.astype(jnp.int32)
    z_pk = z32[0::2] | (z32[1::2] << 16)
    return sc(b12, z_pk, idnb_i.astype(jnp.int32),
              idnb_j.astype(jnp.int32), y2)


# speedup vs baseline: 8.2213x; 1.1470x over previous
import functools

import jax
import jax.numpy as jnp
from jax import lax
from jax.experimental import pallas as pl
from jax.experimental.pallas import tpu as pltpu
from jax.experimental.pallas import tpu_sc as plsc

EMB = 128
LANES = 16
COLS = EMB // LANES

def _swish(x):
    return x / (1.0 + jnp.exp(-x))


def _pack_cols(y):
    u = lax.bitcast_convert_type(y.astype(jnp.bfloat16), jnp.uint16)
    u = u.astype(jnp.uint32)
    lo = jnp.concatenate([u[..., 32 * t:32 * t + 16] for t in range(4)],
                         axis=-1)
    hi = jnp.concatenate([u[..., 32 * t + 16:32 * t + 32] for t in range(4)],
                         axis=-1)
    return lax.bitcast_convert_type(lo | (hi << 16), jnp.float32)


def _tables_body(emb_ref, wp_ref, b12_ref):
    e = emb_ref[...]
    b1 = jnp.dot(e, wp_ref[0:EMB, :], preferred_element_type=jnp.float32)
    b2 = jnp.dot(e, wp_ref[EMB:2 * EMB, :], preferred_element_type=jnp.float32)
    b12_ref[...] = b1[:, None, :] + b2[None, :, :]


def _y_body(rbft_ref, wr_ref, br_ref, w3_ref, b_ref, y2_ref):
    rt = rbft_ref[...]
    t = lax.dot_general(rt, wr_ref[...], (((0,), (0,)), ((), ())),
                        preferred_element_type=jnp.float32)
    t = t + br_ref[...]
    s = jax.nn.swish(t).astype(jnp.bfloat16)
    y = jnp.dot(s, w3_ref[...], preferred_element_type=jnp.float32)
    y = y + b_ref[...]
    y2_ref[...] = pltpu.bitcast(y.astype(jnp.bfloat16), jnp.float32)


def _sc_body(n_chunks, chunk, per_worker, nv_pad, b12_hbm, z_hbm, ii_hbm,
             ij_hbm, y2_hbm, out_hbm, z_v, ii_v, ij_v, zp_v, gb, yb, ob,
             si0, si1, sg0, sg1, so0, so1):
    wid = lax.axis_index("s") * 2 + lax.axis_index("c")
    base = wid * per_worker
    si = (si0, si1)
    sg = (sg0, sg1)
    so = (so0, so1)
    last = n_chunks - 1
    half = chunk // 2

    pltpu.sync_copy(z_hbm, z_v)

    def _off(c):
        loc = jnp.minimum(jnp.minimum(c, last) * chunk, per_worker - chunk)
        return pl.multiple_of(base + loc, 16)

    def fire_idx(c, b):
        off = _off(c)
        pltpu.make_async_copy(ii_hbm.at[pl.ds(off, chunk)], ii_v.at[b],
                              si[b]).start()
        pltpu.make_async_copy(ij_hbm.at[pl.ds(off, chunk)], ij_v.at[b],
                              si[b]).start()

    def wait_idx(b):
        pltpu.make_async_copy(ii_hbm.at[pl.ds(0, chunk)], ii_v.at[b],
                              si[b]).wait()
        pltpu.make_async_copy(ij_hbm.at[pl.ds(0, chunk)], ij_v.at[b],
                              si[b]).wait()

    def fire_y(c, b):
        off2 = pl.multiple_of(_off(c) // 2, 8)
        pltpu.make_async_copy(y2_hbm.at[pl.ds(off2, half)], yb.at[b],
                              sg[b]).start()

    def _z_lookup(idx):
        zr = plsc.load_gather(z_v, [lax.shift_right_logical(idx, 1)])
        odd = lax.bitwise_and(idx, 1)
        return jnp.where(odd == 1, lax.shift_right_logical(zr, 16),
                         lax.bitwise_and(zr, 0xFFFF))

    def index_and_fire_gathers(b):
        @plsc.parallel_loop(0, chunk // LANES)
        def index_body(g):
            sl = pl.ds(g * LANES, LANES)
            zi = _z_lookup(ii_v[b, sl])
            zj = _z_lookup(ij_v[b, sl])
            zp_v[b, sl] = zi * nv_pad + zj

        pltpu.make_async_copy(b12_hbm.at[zp_v.at[b]], gb.at[b], sg[b]).start()

    def wait_gy(b):
        pltpu.make_async_copy(y2_hbm.at[pl.ds(0, half)], yb.at[b],
                              sg[b]).wait()
        pltpu.make_async_copy(b12_hbm.at[zp_v.at[b]], gb.at[b], sg[b]).wait()

    def compute(b):
        @plsc.parallel_loop(0, half)
        def row_body(r):
            e0 = 2 * r
            e1 = e0 + 1
            for q in range(COLS):
                sl = pl.ds(q * LANES, LANES)
                y0, y1 = plsc.unpack(
                    plsc.bitcast(yb[b, r, sl], jnp.bfloat16),
                    format=plsc.PackFormat.INTERLEAVED,
                    preferred_element_type=jnp.float32)
                ob[b, e0, sl] = _swish(gb[b, e0, sl] + y0)
                ob[b, e1, sl] = _swish(gb[b, e1, sl] + y1)

    def fire_out(c, b):
        pltpu.make_async_copy(ob.at[b], out_hbm.at[pl.ds(_off(c), chunk)],
                              so[b]).start()

    def wait_out(b):
        pltpu.make_async_copy(ob.at[b], out_hbm.at[pl.ds(0, chunk)],
                              so[b]).wait()

    fire_idx(0, 0)
    fire_y(0, 0)
    fire_idx(1, 1)
    fire_y(1, 1)
    wait_idx(0)
    index_and_fire_gathers(0)

    n_pairs = n_chunks // 2

    def pair_body(p, _):
        c = 2 * p
        fire_idx(c + 2, 0)
        wait_idx(1)
        index_and_fire_gathers(1)
        wait_gy(0)

        @pl.when(p > 0)
        def _():
            wait_out(0)

        compute(0)
        fire_out(c, 0)
        fire_y(c + 2, 0)
        wait_idx(0)
        index_and_fire_gathers(0)
        wait_gy(1)

        @pl.when(p > 0)
        def _():
            wait_out(1)

        compute(1)
        fire_out(c + 1, 1)
        fire_idx(c + 3, 1)
        fire_y(c + 3, 1)
        return 0

    lax.fori_loop(0, n_pairs, pair_body, 0)

    wait_gy(0)
    wait_out(0)
    compute(0)
    fire_out(last, 0)
    wait_idx(1)
    pltpu.make_async_copy(y2_hbm.at[pl.ds(0, half)], yb.at[1], sg[1]).wait()
    wait_out(1)
    wait_out(0)


def kernel(Z, rbf, idnb_i, idnb_j, embeddings, W_rbf, b_rbf, W, b):
    n_edges = rbf.shape[0]
    n_atoms = Z.shape[0]
    nv = embeddings.shape[0]
    nr = rbf.shape[1]
    nv_pad = ((nv + 7) // 8) * 8
    emb_p = jnp.pad(embeddings, ((0, nv_pad - nv), (0, 0)))
    b12 = pl.pallas_call(
        _tables_body,
        out_shape=jax.ShapeDtypeStruct((nv_pad, nv_pad, EMB), jnp.float32),
    )(emb_p, W[0:2 * EMB])
    b12 = b12.reshape(nv_pad * nv_pad, EMB)

    blk = 6400
    grid = n_edges // blk
    rbf_t = rbf.T
    y2 = pl.pallas_call(
        _y_body,
        grid=(grid,),
        in_specs=[
            pl.BlockSpec((nr, blk), lambda i: (0, i)),
            pl.BlockSpec((nr, EMB), lambda i: (0, 0)),
            pl.BlockSpec((1, EMB), lambda i: (0, 0)),
            pl.BlockSpec((EMB, EMB), lambda i: (0, 0)),
            pl.BlockSpec((1, EMB), lambda i: (0, 0)),
        ],
        out_specs=pl.BlockSpec((blk // 2, EMB), lambda i: (i, 0)),
        out_shape=jax.ShapeDtypeStruct((n_edges // 2, EMB), jnp.float32),
    )(rbf_t, W_rbf, b_rbf.reshape(1, EMB),
      W[2 * EMB:3 * EMB, :].astype(jnp.bfloat16), b.reshape(1, EMB))

    n_workers = 32
    per_worker = n_edges // n_workers
    chunk = 128
    n_chunks = -(-per_worker // chunk)
    if n_chunks % 2 == 0:
        n_chunks += 1

    mesh = plsc.VectorSubcoreMesh(core_axis_name="c", subcore_axis_name="s",
                                  num_cores=2, num_subcores=16)
    sc = pl.kernel(
        functools.partial(_sc_body, n_chunks, chunk, per_worker, nv_pad),
        out_type=jax.ShapeDtypeStruct((n_edges, EMB), jnp.float32),
        mesh=mesh,
        scratch_types=[
            pltpu.VMEM((n_atoms // 2,), jnp.int32),
            pltpu.VMEM((2, chunk), jnp.int32),
            pltpu.VMEM((2, chunk), jnp.int32),
            pltpu.VMEM((2, chunk), jnp.int32),
            pltpu.VMEM((2, chunk, EMB), jnp.float32),
            pltpu.VMEM((2, chunk // 2, EMB), jnp.float32),
            pltpu.VMEM((2, chunk, EMB), jnp.float32),
            pltpu.SemaphoreType.DMA,
            pltpu.SemaphoreType.DMA,
            pltpu.SemaphoreType.DMA,
            pltpu.SemaphoreType.DMA,
            pltpu.SemaphoreType.DMA,
            pltpu.SemaphoreType.DMA,
        ],
        compiler_params=pltpu.CompilerParams(needs_layout_passes=False),
    )
    z32 = Z.astype(jnp.int32)
    z_pk = z32[0::2] | (z32[1::2] << 16)
    return sc(b12, z_pk, idnb_i.astype(jnp.int32),
              idnb_j.astype(jnp.int32), y2)
